# baseline (speedup 1.0000x reference)
import jax
import jax.numpy as jnp
from jax.experimental import pallas as pl
from jax.experimental.pallas import tpu as pltpu

_BB = 128
_NS = 4
_AHEAD = 2


def _body(idx_ref, ppr_hbm, x_ref, w_ref, b_ref, out_ref, enc, buf, sems):
    i = pl.program_id(0)
    nsteps = pl.num_programs(0)
    slot = jax.lax.rem(i, _NS)

    def _issue(step, s):
        def one(k, carry):
            r = idx_ref[step * _BB + k]
            pltpu.make_async_copy(
                ppr_hbm.at[r], buf.at[s, k], sems.at[s]
            ).start(priority=0)
            r2 = idx_ref[step * _BB + k + 1]
            pltpu.make_async_copy(
                ppr_hbm.at[r2], buf.at[s, k + 1], sems.at[s]
            ).start(priority=1)
            return carry

        jax.lax.fori_loop(0, _BB // 2, lambda k, c: one(2 * k, c), 0, unroll=8)

    @pl.when(i == 0)
    def _():
        for s in range(_AHEAD + 1):
            _issue(s, s)
        enc[...] = (
            jnp.dot(x_ref[...], w_ref[...], preferred_element_type=jnp.float32)
            + b_ref[...]
        ).astype(jnp.bfloat16)

    @pl.when(jnp.logical_and(i > 0, i + _AHEAD < nsteps))
    def _():
        _issue(i + _AHEAD, jax.lax.rem(i + _AHEAD, _NS))

    pltpu.make_async_copy(
        ppr_hbm.at[pl.ds(0, _BB)], buf.at[slot], sems.at[slot]
    ).wait()

    out_ref[...] = jnp.dot(
        buf[slot].astype(jnp.bfloat16),
        enc[...],
        preferred_element_type=jnp.float32,
    )


def kernel(X, idx, ppr, W, b):
    n, d = X.shape
    dout = W.shape[1]
    bsz = idx.shape[0]

    out = pl.pallas_call(
        _body,
        grid_spec=pltpu.PrefetchScalarGridSpec(
            num_scalar_prefetch=1,
            grid=(bsz // _BB,),
            in_specs=[
                pl.BlockSpec(memory_space=pltpu.HBM),
                pl.BlockSpec((n, d), lambda i, idx_ref: (0, 0)),
                pl.BlockSpec((d, dout), lambda i, idx_ref: (0, 0)),
                pl.BlockSpec((1, dout), lambda i, idx_ref: (0, 0)),
            ],
            out_specs=pl.BlockSpec((_BB, dout), lambda i, idx_ref: (i, 0)),
            scratch_shapes=[
                pltpu.VMEM((n, dout), jnp.bfloat16),
                pltpu.VMEM((_NS, _BB, n), jnp.float32),
                pltpu.SemaphoreType.DMA((_NS,)),
            ],
        ),
        out_shape=jax.ShapeDtypeStruct((bsz, dout), jnp.float32),
    )(idx.astype(jnp.int32), ppr, X, W, b.reshape(1, dout))
    return out


# speedup vs baseline: 1.0023x; 1.0023x over previous
import jax
import jax.numpy as jnp
from jax.experimental import pallas as pl
from jax.experimental.pallas import tpu as pltpu

_BB = 128
_NS = 4
_AHEAD = 2


def _body(rows_ref, cnt_ref, ppr_hbm, x_ref, w_ref, b_ref, uidx_ref,
          out_ref, enc, buf, sems):
    i = pl.program_id(0)
    nsteps = pl.num_programs(0)
    slot = jax.lax.rem(i, _NS)
    nbytes_row = buf.shape[2] * 4

    def _issue(step, s):
        cnt = cnt_ref[step]

        def one(k, carry):
            @pl.when(k < cnt)
            def _():
                r = rows_ref[step * _BB + k]
                pltpu.make_async_copy(
                    ppr_hbm.at[r], buf.at[s, k], sems.at[s]
                ).start()
            return carry

        jax.lax.fori_loop(0, _BB, one, 0, unroll=16)

    @pl.when(i == 0)
    def _():
        for s in range(_AHEAD + 1):
            _issue(s, s)
        enc[...] = (
            jnp.dot(x_ref[...], w_ref[...], preferred_element_type=jnp.float32)
            + b_ref[...]
        ).astype(jnp.bfloat16)

    @pl.when(jnp.logical_and(i > 0, i + _AHEAD < nsteps))
    def _():
        _issue(i + _AHEAD, jax.lax.rem(i + _AHEAD, _NS))

    def _waitone(k, carry):
        @pl.when(k < cnt_ref[i])
        def _():
            pltpu.make_async_copy(
                ppr_hbm.at[0], buf.at[slot, k], sems.at[slot]
            ).wait()
        return carry

    jax.lax.fori_loop(0, _BB, _waitone, 0, unroll=16)

    out_u = jnp.dot(
        buf[slot].astype(jnp.bfloat16),
        enc[...],
        preferred_element_type=jnp.float32,
    )
    out_u = jnp.where(jnp.isfinite(out_u), out_u, 0.0)
    iota0 = jax.lax.broadcasted_iota(jnp.int32, (_BB, _BB), 0)
    pt = (uidx_ref[0] == iota0).astype(jnp.float32)
    out_ref[...] = jax.lax.dot_general(
        pt, out_u, (((0,), (0,)), ((), ())),
        preferred_element_type=jnp.float32,
    )


def kernel(X, idx, ppr, W, b):
    n, d = X.shape
    dout = W.shape[1]
    bsz = idx.shape[0]
    nsteps = bsz // _BB

    idx32 = idx.astype(jnp.int32)
    perm = jnp.argsort(idx32)
    sid = idx32[perm]
    j = jnp.arange(bsz, dtype=jnp.int32)
    new = (sid != jnp.roll(sid, 1)) | (j % _BB == 0)
    uidx = jnp.cumsum(new.reshape(nsteps, _BB), axis=1).astype(jnp.int32) - 1
    cnt = (uidx[:, -1] + 1).astype(jnp.int32)
    rows = jnp.zeros((nsteps, _BB), jnp.int32)
    rows = rows.at[j // _BB, uidx.reshape(-1)].set(sid)

    out_sorted = pl.pallas_call(
        _body,
        grid_spec=pltpu.PrefetchScalarGridSpec(
            num_scalar_prefetch=2,
            grid=(nsteps,),
            in_specs=[
                pl.BlockSpec(memory_space=pltpu.HBM),
                pl.BlockSpec((n, d), lambda i, *_: (0, 0)),
                pl.BlockSpec((d, dout), lambda i, *_: (0, 0)),
                pl.BlockSpec((1, dout), lambda i, *_: (0, 0)),
                pl.BlockSpec((1, 1, _BB), lambda i, *_: (i, 0, 0)),
            ],
            out_specs=pl.BlockSpec((_BB, dout), lambda i, *_: (i, 0)),
            scratch_shapes=[
                pltpu.VMEM((n, dout), jnp.bfloat16),
                pltpu.VMEM((_NS, _BB, n), jnp.float32),
                pltpu.SemaphoreType.DMA((_NS,)),
            ],
        ),
        out_shape=jax.ShapeDtypeStruct((bsz, dout), jnp.float32),
    )(rows.reshape(-1), cnt, ppr, X, W, b.reshape(1, dout),
      uidx.reshape(nsteps, 1, _BB))
    out = jnp.zeros((bsz, dout), jnp.float32).at[perm].set(out_sorted)
    return out


# speedup vs baseline: 2.4785x; 2.4728x over previous
import jax
import jax.numpy as jnp
from jax.experimental import pallas as pl
from jax.experimental.pallas import tpu as pltpu

_BB = 256
_NS = 3
_AHEAD = 1


def _body(idx_ref, ppr_hbm, x_ref, w_ref, b_ref, out_ref, enc, buf, sems):
    i = pl.program_id(0)
    nsteps = pl.num_programs(0)
    slot = jax.lax.rem(i, _NS)

    def _issue(step, s):
        def one(k, carry):
            r = idx_ref[step * _BB + k]
            pltpu.make_async_copy(
                ppr_hbm.at[r], buf.at[s, k], sems.at[s]
            ).start()
            return carry

        jax.lax.fori_loop(0, _BB, one, 0, unroll=16)

    @pl.when(i == 0)
    def _():
        for s in range(_AHEAD + 1):
            _issue(s, s)
        enc[...] = (
            jnp.dot(x_ref[...], w_ref[...], preferred_element_type=jnp.float32)
            + b_ref[...]
        ).astype(jnp.bfloat16)

    @pl.when(jnp.logical_and(i > 0, i + _AHEAD < nsteps))
    def _():
        _issue(i + _AHEAD, jax.lax.rem(i + _AHEAD, _NS))

    pltpu.make_async_copy(
        ppr_hbm.at[pl.ds(0, _BB)], buf.at[slot], sems.at[slot]
    ).wait()

    out_ref[...] = jnp.dot(
        buf[slot].astype(jnp.bfloat16),
        enc[...],
        preferred_element_type=jnp.float32,
    )


def kernel(X, idx, ppr, W, b):
    n, d = X.shape
    dout = W.shape[1]
    bsz = idx.shape[0]

    out = pl.pallas_call(
        _body,
        grid_spec=pltpu.PrefetchScalarGridSpec(
            num_scalar_prefetch=1,
            grid=(bsz // _BB,),
            in_specs=[
                pl.BlockSpec(memory_space=pltpu.HBM),
                pl.BlockSpec((n, d), lambda i, idx_ref: (0, 0)),
                pl.BlockSpec((d, dout), lambda i, idx_ref: (0, 0)),
                pl.BlockSpec((1, dout), lambda i, idx_ref: (0, 0)),
            ],
            out_specs=pl.BlockSpec((_BB, dout), lambda i, idx_ref: (i, 0)),
            scratch_shapes=[
                pltpu.VMEM((n, dout), jnp.bfloat16),
                pltpu.VMEM((_NS, _BB, n), jnp.float32),
                pltpu.SemaphoreType.DMA((_NS,)),
            ],
        ),
        out_shape=jax.ShapeDtypeStruct((bsz, dout), jnp.float32),
    )(idx.astype(jnp.int32), ppr, X, W, b.reshape(1, dout))
    return out


# speedup vs baseline: 2.5472x; 1.0277x over previous
import jax
import jax.numpy as jnp
from jax.experimental import pallas as pl
from jax.experimental.pallas import tpu as pltpu

_BB = 256
_NS = 4
_AHEAD = 2


def _body(idx_ref, ppr_hbm, x_ref, w_ref, b_ref, out_ref, enc, buf, sems):
    i = pl.program_id(0)
    nsteps = pl.num_programs(0)
    slot = jax.lax.rem(i, _NS)

    def _issue(step, s):
        def one(k, carry):
            r = idx_ref[step * _BB + k]
            pltpu.make_async_copy(
                ppr_hbm.at[r], buf.at[s, k], sems.at[s]
            ).start()
            return carry

        jax.lax.fori_loop(0, _BB, one, 0, unroll=16)

    @pl.when(i == 0)
    def _():
        for s in range(_AHEAD + 1):
            _issue(s, s)
        enc[...] = (
            jnp.dot(x_ref[...], w_ref[...], preferred_element_type=jnp.float32)
            + b_ref[...]
        ).astype(jnp.bfloat16)

    @pl.when(jnp.logical_and(i > 0, i + _AHEAD < nsteps))
    def _():
        _issue(i + _AHEAD, jax.lax.rem(i + _AHEAD, _NS))

    pltpu.make_async_copy(
        ppr_hbm.at[pl.ds(0, _BB)], buf.at[slot], sems.at[slot]
    ).wait()

    out_ref[...] = jnp.dot(
        buf[slot].astype(jnp.bfloat16),
        enc[...],
        preferred_element_type=jnp.float32,
    )


def kernel(X, idx, ppr, W, b):
    n, d = X.shape
    dout = W.shape[1]
    bsz = idx.shape[0]

    out = pl.pallas_call(
        _body,
        grid_spec=pltpu.PrefetchScalarGridSpec(
            num_scalar_prefetch=1,
            grid=(bsz // _BB,),
            in_specs=[
                pl.BlockSpec(memory_space=pltpu.HBM),
                pl.BlockSpec((n, d), lambda i, idx_ref: (0, 0)),
                pl.BlockSpec((d, dout), lambda i, idx_ref: (0, 0)),
                pl.BlockSpec((1, dout), lambda i, idx_ref: (0, 0)),
            ],
            out_specs=pl.BlockSpec((_BB, dout), lambda i, idx_ref: (i, 0)),
            scratch_shapes=[
                pltpu.VMEM((n, dout), jnp.bfloat16),
                pltpu.VMEM((_NS, _BB, n), jnp.float32),
                pltpu.SemaphoreType.DMA((_NS,)),
            ],
        ),
        out_shape=jax.ShapeDtypeStruct((bsz, dout), jnp.float32),
    )(idx.astype(jnp.int32), ppr, X, W, b.reshape(1, dout))
    return out
